# Initial kernel scaffold; baseline (speedup 1.0000x reference)
import jax
import jax.numpy as jnp
from jax.experimental import pallas as pl


def kernel(x, edge_index, edge_attr, batch, A1, b1, root1, bias1, A2, b2, root2, bias2):
    raise NotImplementedError("write your pallas kernel here")


# speedup vs baseline: 1.4411x; 1.4411x over previous
import functools

import jax
import jax.numpy as jnp
from jax import lax
from jax.experimental import pallas as pl
from jax.experimental.pallas import tpu as pltpu
from jax.experimental.pallas import tpu_sc as plsc

_INFO = plsc.get_sparse_core_info()
_NC = _INFO.num_cores
_NS = _INFO.num_subcores
_NW = _NC * _NS
_LC = 128


def _make_gather(n_nodes, feat, n_edges):
    nchunk = n_edges // _LC
    steps = (nchunk + _NW - 1) // _NW
    mesh = plsc.VectorSubcoreMesh(core_axis_name="c", subcore_axis_name="s")

    @functools.partial(
        pl.kernel,
        out_type=jax.ShapeDtypeStruct((n_edges, feat), jnp.float32),
        mesh=mesh,
        scratch_types=[
            pltpu.VMEM((_LC,), jnp.int32),
            pltpu.VMEM((_LC, feat), jnp.float32),
            pltpu.SemaphoreType.DMA,
        ],
    )
    def gather(table_hbm, src_hbm, out_hbm, idx_v, rows_v, sem):
        cid = lax.axis_index("c")
        sid = lax.axis_index("s")
        wid = sid * _NC + cid

        def body(i, carry):
            c = wid + i * _NW

            @pl.when(c < nchunk)
            def _():
                pltpu.sync_copy(src_hbm.at[c, 0], idx_v)
                pltpu.async_copy(table_hbm.at[idx_v], rows_v, sem).wait()
                pltpu.sync_copy(rows_v, out_hbm.at[pl.ds(c * _LC, _LC)])

            return carry

        lax.fori_loop(0, steps, body, 0)

    return gather


def _pad_rows(n):
    q = 8 * _NS
    return (n + q - 1) // q * q


def _make_scatter(n_nodes, n_edges):
    nchunk = n_edges // _LC
    steps = (nchunk + _NW - 1) // _NW
    npad = _pad_rows(n_nodes)
    rpt = npad // _NS
    mesh = plsc.VectorSubcoreMesh(core_axis_name="c", subcore_axis_name="s")

    @functools.partial(
        pl.kernel,
        out_type=jax.ShapeDtypeStruct((_NC, npad, 128), jnp.float32),
        mesh=mesh,
        scratch_types=[
            pltpu.VMEM((_LC,), jnp.int32),
            pltpu.VMEM((_LC, 128), jnp.float32),
            pltpu.VMEM_SHARED((npad, 128), jnp.float32),
            pltpu.SemaphoreType.DMA,
        ],
    )
    def scatter(rows_hbm, dst_hbm, zeros_hbm, out_hbm, idx_v, rows_v, agg_sh, sem):
        cid = lax.axis_index("c")
        sid = lax.axis_index("s")
        wid = sid * _NC + cid

        pltpu.sync_copy(zeros_hbm.at[pl.ds(sid * rpt, rpt)],
                        agg_sh.at[pl.ds(sid * rpt, rpt)])
        plsc.subcore_barrier()

        def body(i, carry):
            c = wid + i * _NW

            @pl.when(c < nchunk)
            def _():
                pltpu.sync_copy(dst_hbm.at[c, 0], idx_v)
                pltpu.sync_copy(rows_hbm.at[pl.ds(c * _LC, _LC)], rows_v)
                pltpu.sync_copy(rows_v, agg_sh.at[idx_v], add=True)

            return carry

        lax.fori_loop(0, steps, body, 0)
        plsc.subcore_barrier()

        pltpu.sync_copy(agg_sh.at[pl.ds(sid * rpt, rpt)],
                        out_hbm.at[cid, pl.ds(sid * rpt, rpt)])

    return scatter


def _edge_messages(a, xs, A, b, fin, eb=64):
    e = xs.shape[0]
    fout = A.shape[1]

    def body(a_ref, xs_ref, A_ref, b_ref, o_ref):
        av = a_ref[...]
        w = jnp.maximum(av[:, :, None] * A_ref[...][None, :, :]
                        + b_ref[...][None, :, :], 0.0)
        msg = jnp.sum(xs_ref[...][:, :fin, None] * w, axis=1)
        o_ref[...] = jnp.concatenate(
            [msg,
             jnp.ones((msg.shape[0], 16), jnp.float32),
             jnp.zeros((msg.shape[0], 128 - fout - 16), jnp.float32)], axis=1)

    return pl.pallas_call(
        body,
        grid=(e // eb,),
        in_specs=[
            pl.BlockSpec((eb, 1), lambda i: (i, 0)),
            pl.BlockSpec((eb, xs.shape[1]), lambda i: (i, 0)),
            pl.BlockSpec((fin, fout), lambda i: (0, 0)),
            pl.BlockSpec((fin, fout), lambda i: (0, 0)),
        ],
        out_specs=pl.BlockSpec((eb, 128), lambda i: (i, 0)),
        out_shape=jax.ShapeDtypeStruct((e, 128), jnp.float32),
    )(a, xs, A, b)


def _combine(x, root, p0, p1, bias8, nb=1000):
    n, fin = x.shape
    fout = root.shape[1]

    def body(x_ref, r_ref, p0_ref, p1_ref, b_ref, o_ref):
        p = p0_ref[...] + p1_ref[...]
        agg = p[:, :fout]
        deg = jnp.maximum(p[:, fout:fout + 1], 1.0)
        h = jnp.dot(x_ref[...][:, :fin], r_ref[...],
                    preferred_element_type=jnp.float32)
        val = jnp.maximum(h + agg / deg + b_ref[0:1, :], 0.0)
        o_ref[...] = jnp.concatenate(
            [val, jnp.zeros((val.shape[0], 128 - fout), jnp.float32)], axis=1)

    return pl.pallas_call(
        body,
        grid=(n // nb,),
        in_specs=[
            pl.BlockSpec((nb, x.shape[1]), lambda i: (i, 0)),
            pl.BlockSpec((fin, fout), lambda i: (0, 0)),
            pl.BlockSpec((nb, 128), lambda i: (i, 0)),
            pl.BlockSpec((nb, 128), lambda i: (i, 0)),
            pl.BlockSpec((8, fout), lambda i: (0, 0)),
        ],
        out_specs=pl.BlockSpec((nb, 128), lambda i: (i, 0)),
        out_shape=jax.ShapeDtypeStruct((n, 128), jnp.float32),
    )(x, root, p0, p1, bias8)


def _pool(batch3, h, nb=1000):
    n = h.shape[0]
    grid = n // nb

    def body(b_ref, h_ref, o_ref, s_acc, c_acc):
        i = pl.program_id(0)

        @pl.when(i == 0)
        def _():
            s_acc[...] = jnp.zeros_like(s_acc)
            c_acc[...] = jnp.zeros_like(c_acc)

        bb = b_ref[0, 0, :]
        gid = lax.broadcasted_iota(jnp.int32, (16, nb), 0)
        m = (gid == bb[None, :]).astype(jnp.float32)
        s_acc[...] += jnp.dot(m, h_ref[...],
                              preferred_element_type=jnp.float32)
        c_acc[...] += jnp.broadcast_to(
            jnp.sum(m, axis=1, keepdims=True), c_acc.shape)

        @pl.when(i == grid - 1)
        def _():
            o_ref[...] = s_acc[...] / jnp.maximum(c_acc[...], 1.0)

    return pl.pallas_call(
        body,
        grid=(grid,),
        in_specs=[
            pl.BlockSpec((1, 1, nb), lambda i: (i, 0, 0)),
            pl.BlockSpec((nb, 128), lambda i: (i, 0)),
        ],
        out_specs=pl.BlockSpec((16, 128), lambda i: (0, 0)),
        out_shape=jax.ShapeDtypeStruct((16, 128), jnp.float32),
        scratch_shapes=[
            pltpu.VMEM((16, 128), jnp.float32),
            pltpu.VMEM((16, 128), jnp.float32),
        ],
    )(batch3, h)


def kernel(x, edge_index, edge_attr, batch, A1, b1, root1, bias1,
           A2, b2, root2, bias2):
    n, fin = x.shape
    e = edge_attr.shape[0]
    fmid = root1.shape[1]

    src = edge_index[0].reshape(e // _LC, 1, _LC)
    dst = edge_index[1].reshape(e // _LC, 1, _LC)
    npad = _pad_rows(n)
    A1m = A1.reshape(fin, fmid)
    b1m = b1.reshape(fin, fmid)
    A2m = A2.reshape(fmid, fmid)
    b2m = b2.reshape(fmid, fmid)
    bias1_8 = jnp.broadcast_to(bias1.reshape(1, fmid), (8, fmid))
    bias2_8 = jnp.broadcast_to(bias2.reshape(1, fmid), (8, fmid))
    zeros128 = jnp.zeros((npad, 128), jnp.float32)
    batch3 = batch.reshape(10, 1, n // 10)

    gather = _make_gather(n, 128, e)
    scat = _make_scatter(n, e)

    xs = gather(x, src)
    msgs1 = _edge_messages(edge_attr, xs, A1m, b1m, fin)
    p1 = scat(msgs1, dst, zeros128)
    h1p = _combine(x, root1, p1[0, :n], p1[1, :n], bias1_8)

    hs = gather(h1p, src)
    msgs2 = _edge_messages(edge_attr, hs, A2m, b2m, fmid)
    p2 = scat(msgs2, dst, zeros128)
    h2p = _combine(h1p, root2, p2[0, :n], p2[1, :n], bias2_8)

    pooled = _pool(batch3, h2p)
    return pooled[:10, :fmid]


# speedup vs baseline: 1.5524x; 1.0773x over previous
import functools

import jax
import jax.numpy as jnp
from jax import lax
from jax.experimental import pallas as pl
from jax.experimental.pallas import tpu as pltpu
from jax.experimental.pallas import tpu_sc as plsc

_INFO = plsc.get_sparse_core_info()
_NC = _INFO.num_cores
_NS = _INFO.num_subcores
_NW = _NC * _NS
_LC = 128


def _make_gather(n_nodes, feat, n_edges):
    nchunk = n_edges // _LC
    steps = (nchunk + _NW - 1) // _NW
    mesh = plsc.VectorSubcoreMesh(core_axis_name="c", subcore_axis_name="s")

    @functools.partial(
        pl.kernel,
        out_type=jax.ShapeDtypeStruct((n_edges, feat), jnp.float32),
        mesh=mesh,
        scratch_types=[
            pltpu.VMEM((_LC,), jnp.int32),
            pltpu.VMEM((_LC, feat), jnp.float32),
            pltpu.SemaphoreType.DMA,
        ],
    )
    def gather(table_hbm, src_hbm, out_hbm, idx_v, rows_v, sem):
        cid = lax.axis_index("c")
        sid = lax.axis_index("s")
        wid = sid * _NC + cid

        def body(i, carry):
            c = wid + i * _NW

            @pl.when(c < nchunk)
            def _():
                pltpu.sync_copy(src_hbm.at[c, 0], idx_v)
                pltpu.async_copy(table_hbm.at[idx_v], rows_v, sem).wait()
                pltpu.sync_copy(rows_v, out_hbm.at[pl.ds(c * _LC, _LC)])

            return carry

        lax.fori_loop(0, steps, body, 0)

    return gather


def _pad_rows(n):
    q = 8 * _NS
    return (n + q - 1) // q * q


def _make_scatter(n_nodes, n_edges):
    nchunk = n_edges // _LC
    steps = (nchunk + _NW - 1) // _NW
    npad = _pad_rows(n_nodes)
    rpt = npad // _NS
    mesh = plsc.VectorSubcoreMesh(core_axis_name="c", subcore_axis_name="s")

    @functools.partial(
        pl.kernel,
        out_type=jax.ShapeDtypeStruct((_NC, npad, 128), jnp.float32),
        mesh=mesh,
        scratch_types=[
            pltpu.VMEM((_LC,), jnp.int32),
            pltpu.VMEM((_LC, 128), jnp.float32),
            pltpu.VMEM_SHARED((npad, 128), jnp.float32),
            pltpu.SemaphoreType.DMA,
        ],
    )
    def scatter(rows_hbm, dst_hbm, zeros_hbm, out_hbm, idx_v, rows_v, agg_sh, sem):
        cid = lax.axis_index("c")
        sid = lax.axis_index("s")
        wid = sid * _NC + cid

        pltpu.sync_copy(zeros_hbm.at[pl.ds(sid * rpt, rpt)],
                        agg_sh.at[pl.ds(sid * rpt, rpt)])
        plsc.subcore_barrier()

        def body(i, carry):
            c = wid + i * _NW

            @pl.when(c < nchunk)
            def _():
                pltpu.sync_copy(dst_hbm.at[c, 0], idx_v)
                pltpu.sync_copy(rows_hbm.at[pl.ds(c * _LC, _LC)], rows_v)
                pltpu.sync_copy(rows_v, agg_sh.at[idx_v], add=True)

            return carry

        lax.fori_loop(0, steps, body, 0)
        plsc.subcore_barrier()

        pltpu.sync_copy(agg_sh.at[pl.ds(sid * rpt, rpt)],
                        out_hbm.at[cid, pl.ds(sid * rpt, rpt)])

    return scatter


def _edge_messages(a, xs, A, b, fin, eb=256):
    e = xs.shape[0]
    fout = A.shape[1]

    def body(a_ref, xs_ref, A_ref, b_ref, o_ref):
        av = a_ref[...]
        w = jnp.maximum(av[:, :, None] * A_ref[...][None, :, :]
                        + b_ref[...][None, :, :], 0.0)
        msg = jnp.sum(xs_ref[...][:, :fin, None] * w, axis=1)
        o_ref[...] = jnp.concatenate(
            [msg,
             jnp.ones((msg.shape[0], 16), jnp.float32),
             jnp.zeros((msg.shape[0], 128 - fout - 16), jnp.float32)], axis=1)

    return pl.pallas_call(
        body,
        grid=(e // eb,),
        in_specs=[
            pl.BlockSpec((eb, 1), lambda i: (i, 0)),
            pl.BlockSpec((eb, xs.shape[1]), lambda i: (i, 0)),
            pl.BlockSpec((fin, fout), lambda i: (0, 0)),
            pl.BlockSpec((fin, fout), lambda i: (0, 0)),
        ],
        out_specs=pl.BlockSpec((eb, 128), lambda i: (i, 0)),
        out_shape=jax.ShapeDtypeStruct((e, 128), jnp.float32),
    )(a, xs, A, b)


def _combine(x, root, p0, p1, bias8, nb=1000):
    n, fin = x.shape
    fout = root.shape[1]

    def body(x_ref, r_ref, p0_ref, p1_ref, b_ref, o_ref):
        p = p0_ref[...] + p1_ref[...]
        agg = p[:, :fout]
        deg = jnp.maximum(p[:, fout:fout + 1], 1.0)
        h = jnp.dot(x_ref[...][:, :fin], r_ref[...],
                    preferred_element_type=jnp.float32)
        val = jnp.maximum(h + agg / deg + b_ref[0:1, :], 0.0)
        o_ref[...] = jnp.concatenate(
            [val, jnp.zeros((val.shape[0], 128 - fout), jnp.float32)], axis=1)

    return pl.pallas_call(
        body,
        grid=(n // nb,),
        in_specs=[
            pl.BlockSpec((nb, x.shape[1]), lambda i: (i, 0)),
            pl.BlockSpec((fin, fout), lambda i: (0, 0)),
            pl.BlockSpec((nb, 128), lambda i: (i, 0)),
            pl.BlockSpec((nb, 128), lambda i: (i, 0)),
            pl.BlockSpec((8, fout), lambda i: (0, 0)),
        ],
        out_specs=pl.BlockSpec((nb, 128), lambda i: (i, 0)),
        out_shape=jax.ShapeDtypeStruct((n, 128), jnp.float32),
    )(x, root, p0, p1, bias8)


def _pool(batch3, h, nb=1000):
    n = h.shape[0]
    grid = n // nb

    def body(b_ref, h_ref, o_ref, s_acc, c_acc):
        i = pl.program_id(0)

        @pl.when(i == 0)
        def _():
            s_acc[...] = jnp.zeros_like(s_acc)
            c_acc[...] = jnp.zeros_like(c_acc)

        bb = b_ref[0, 0, :]
        gid = lax.broadcasted_iota(jnp.int32, (16, nb), 0)
        m = (gid == bb[None, :]).astype(jnp.float32)
        s_acc[...] += jnp.dot(m, h_ref[...],
                              preferred_element_type=jnp.float32)
        c_acc[...] += jnp.broadcast_to(
            jnp.sum(m, axis=1, keepdims=True), c_acc.shape)

        @pl.when(i == grid - 1)
        def _():
            o_ref[...] = s_acc[...] / jnp.maximum(c_acc[...], 1.0)

    return pl.pallas_call(
        body,
        grid=(grid,),
        in_specs=[
            pl.BlockSpec((1, 1, nb), lambda i: (i, 0, 0)),
            pl.BlockSpec((nb, 128), lambda i: (i, 0)),
        ],
        out_specs=pl.BlockSpec((16, 128), lambda i: (0, 0)),
        out_shape=jax.ShapeDtypeStruct((16, 128), jnp.float32),
        scratch_shapes=[
            pltpu.VMEM((16, 128), jnp.float32),
            pltpu.VMEM((16, 128), jnp.float32),
        ],
    )(batch3, h)


def kernel(x, edge_index, edge_attr, batch, A1, b1, root1, bias1,
           A2, b2, root2, bias2):
    n, fin = x.shape
    e = edge_attr.shape[0]
    fmid = root1.shape[1]

    src = edge_index[0].reshape(e // _LC, 1, _LC)
    dst = edge_index[1].reshape(e // _LC, 1, _LC)
    npad = _pad_rows(n)
    A1m = A1.reshape(fin, fmid)
    b1m = b1.reshape(fin, fmid)
    A2m = A2.reshape(fmid, fmid)
    b2m = b2.reshape(fmid, fmid)
    bias1_8 = jnp.broadcast_to(bias1.reshape(1, fmid), (8, fmid))
    bias2_8 = jnp.broadcast_to(bias2.reshape(1, fmid), (8, fmid))
    zeros128 = jnp.zeros((npad, 128), jnp.float32)
    batch3 = batch.reshape(10, 1, n // 10)

    gather = _make_gather(n, 128, e)
    scat = _make_scatter(n, e)

    xs = gather(x, src)
    msgs1 = _edge_messages(edge_attr, xs, A1m, b1m, fin)
    p1 = scat(msgs1, dst, zeros128)
    h1p = _combine(x, root1, p1[0, :n], p1[1, :n], bias1_8)

    hs = gather(h1p, src)
    msgs2 = _edge_messages(edge_attr, hs, A2m, b2m, fmid)
    p2 = scat(msgs2, dst, zeros128)
    h2p = _combine(h1p, root2, p2[0, :n], p2[1, :n], bias2_8)

    pooled = _pool(batch3, h2p)
    return pooled[:10, :fmid]


# speedup vs baseline: 2.8185x; 1.8155x over previous
import functools

import jax
import jax.numpy as jnp
from jax import lax
from jax.experimental import pallas as pl
from jax.experimental.pallas import tpu as pltpu
from jax.experimental.pallas import tpu_sc as plsc

_INFO = plsc.get_sparse_core_info()
_NC = _INFO.num_cores
_NS = _INFO.num_subcores
_NW = _NC * _NS
_LC = 128


def _make_gather(n_nodes, feat, n_edges):
    nchunk = n_edges // _LC
    steps = (nchunk + _NW - 1) // _NW
    mesh = plsc.VectorSubcoreMesh(core_axis_name="c", subcore_axis_name="s")

    @functools.partial(
        pl.kernel,
        out_type=jax.ShapeDtypeStruct((n_edges, feat), jnp.float32),
        mesh=mesh,
        scratch_types=[
            pltpu.VMEM((_LC,), jnp.int32),
            pltpu.VMEM((_LC, feat), jnp.float32),
            pltpu.SemaphoreType.DMA,
        ],
    )
    def gather(table_hbm, src_hbm, out_hbm, idx_v, rows_v, sem):
        cid = lax.axis_index("c")
        sid = lax.axis_index("s")
        wid = sid * _NC + cid

        def body(i, carry):
            c = wid + i * _NW

            @pl.when(c < nchunk)
            def _():
                pltpu.sync_copy(src_hbm.at[c, 0], idx_v)
                pltpu.async_copy(table_hbm.at[idx_v], rows_v, sem).wait()
                pltpu.sync_copy(rows_v, out_hbm.at[pl.ds(c * _LC, _LC)])

            return carry

        lax.fori_loop(0, steps, body, 0)

    return gather


def _pad_rows(n):
    q = 8 * _NS
    return (n + q - 1) // q * q


def _make_scatter(n_nodes, n_edges):
    nchunk = n_edges // _LC
    steps = (nchunk + _NW - 1) // _NW
    npad = _pad_rows(n_nodes)
    rpt = npad // _NS
    mesh = plsc.VectorSubcoreMesh(core_axis_name="c", subcore_axis_name="s")

    @functools.partial(
        pl.kernel,
        out_type=jax.ShapeDtypeStruct((_NC, npad, 128), jnp.float32),
        mesh=mesh,
        scratch_types=[
            pltpu.VMEM((_LC,), jnp.int32),
            pltpu.VMEM((_LC, 128), jnp.float32),
            pltpu.VMEM_SHARED((npad, 128), jnp.float32),
            pltpu.SemaphoreType.DMA,
        ],
    )
    def scatter(rows_hbm, dst_hbm, zeros_hbm, out_hbm, idx_v, rows_v, agg_sh, sem):
        cid = lax.axis_index("c")
        sid = lax.axis_index("s")
        wid = sid * _NC + cid

        pltpu.sync_copy(zeros_hbm.at[pl.ds(sid * rpt, rpt)],
                        agg_sh.at[pl.ds(sid * rpt, rpt)])
        plsc.subcore_barrier()

        def body(i, carry):
            c = wid + i * _NW

            @pl.when(c < nchunk)
            def _():
                pltpu.sync_copy(dst_hbm.at[c, 0], idx_v)
                pltpu.sync_copy(rows_hbm.at[pl.ds(c * _LC, _LC)], rows_v)
                pltpu.sync_copy(rows_v, agg_sh.at[idx_v], add=True)

            return carry

        lax.fori_loop(0, steps, body, 0)
        plsc.subcore_barrier()

        pltpu.sync_copy(agg_sh.at[pl.ds(sid * rpt, rpt)],
                        out_hbm.at[cid, pl.ds(sid * rpt, rpt)])

    return scatter


def _edge_messages(a3, xs, At3, bt3, fin, eb=128):
    e = xs.shape[0]
    fout = At3.shape[0]

    def body(a_ref, xs_ref, A_ref, b_ref, o_ref):
        av = a_ref[...]
        xst = xs_ref[...][:, :fin].T
        w = jnp.maximum(av * A_ref[...] + b_ref[...], 0.0)
        msg_t = jnp.sum(xst[None, :, :] * w, axis=1)
        full = jnp.concatenate(
            [msg_t,
             jnp.ones((16, eb), jnp.float32),
             jnp.zeros((128 - fout - 16, eb), jnp.float32)], axis=0)
        o_ref[...] = full.T

    return pl.pallas_call(
        body,
        grid=(e // eb,),
        in_specs=[
            pl.BlockSpec((1, 1, eb), lambda i: (i, 0, 0)),
            pl.BlockSpec((eb, xs.shape[1]), lambda i: (i, 0)),
            pl.BlockSpec((fout, fin, eb), lambda i: (0, 0, 0)),
            pl.BlockSpec((fout, fin, eb), lambda i: (0, 0, 0)),
        ],
        out_specs=pl.BlockSpec((eb, 128), lambda i: (i, 0)),
        out_shape=jax.ShapeDtypeStruct((e, 128), jnp.float32),
    )(a3, xs, At3, bt3)


def _combine(x, root, p0, p1, bias8, nb=1000):
    n, fin = x.shape
    fout = root.shape[1]

    def body(x_ref, r_ref, p0_ref, p1_ref, b_ref, o_ref):
        p = p0_ref[...] + p1_ref[...]
        agg = p[:, :fout]
        deg = jnp.maximum(p[:, fout:fout + 1], 1.0)
        h = jnp.dot(x_ref[...][:, :fin], r_ref[...],
                    preferred_element_type=jnp.float32)
        val = jnp.maximum(h + agg / deg + b_ref[0:1, :], 0.0)
        o_ref[...] = jnp.concatenate(
            [val, jnp.zeros((val.shape[0], 128 - fout), jnp.float32)], axis=1)

    return pl.pallas_call(
        body,
        grid=(n // nb,),
        in_specs=[
            pl.BlockSpec((nb, x.shape[1]), lambda i: (i, 0)),
            pl.BlockSpec((fin, fout), lambda i: (0, 0)),
            pl.BlockSpec((nb, 128), lambda i: (i, 0)),
            pl.BlockSpec((nb, 128), lambda i: (i, 0)),
            pl.BlockSpec((8, fout), lambda i: (0, 0)),
        ],
        out_specs=pl.BlockSpec((nb, 128), lambda i: (i, 0)),
        out_shape=jax.ShapeDtypeStruct((n, 128), jnp.float32),
    )(x, root, p0, p1, bias8)


def _pool(batch3, h, nb=1000):
    n = h.shape[0]
    grid = n // nb

    def body(b_ref, h_ref, o_ref, s_acc, c_acc):
        i = pl.program_id(0)

        @pl.when(i == 0)
        def _():
            s_acc[...] = jnp.zeros_like(s_acc)
            c_acc[...] = jnp.zeros_like(c_acc)

        bb = b_ref[0, 0, :]
        gid = lax.broadcasted_iota(jnp.int32, (16, nb), 0)
        m = (gid == bb[None, :]).astype(jnp.float32)
        s_acc[...] += jnp.dot(m, h_ref[...],
                              preferred_element_type=jnp.float32)
        c_acc[...] += jnp.broadcast_to(
            jnp.sum(m, axis=1, keepdims=True), c_acc.shape)

        @pl.when(i == grid - 1)
        def _():
            o_ref[...] = s_acc[...] / jnp.maximum(c_acc[...], 1.0)

    return pl.pallas_call(
        body,
        grid=(grid,),
        in_specs=[
            pl.BlockSpec((1, 1, nb), lambda i: (i, 0, 0)),
            pl.BlockSpec((nb, 128), lambda i: (i, 0)),
        ],
        out_specs=pl.BlockSpec((16, 128), lambda i: (0, 0)),
        out_shape=jax.ShapeDtypeStruct((16, 128), jnp.float32),
        scratch_shapes=[
            pltpu.VMEM((16, 128), jnp.float32),
            pltpu.VMEM((16, 128), jnp.float32),
        ],
    )(batch3, h)


def kernel(x, edge_index, edge_attr, batch, A1, b1, root1, bias1,
           A2, b2, root2, bias2):
    n, fin = x.shape
    e = edge_attr.shape[0]
    fmid = root1.shape[1]

    src = edge_index[0].reshape(e // _LC, 1, _LC)
    dst = edge_index[1].reshape(e // _LC, 1, _LC)
    npad = _pad_rows(n)
    A1m = A1.reshape(fin, fmid)
    b1m = b1.reshape(fin, fmid)
    A2m = A2.reshape(fmid, fmid)
    b2m = b2.reshape(fmid, fmid)
    bias1_8 = jnp.broadcast_to(bias1.reshape(1, fmid), (8, fmid))
    bias2_8 = jnp.broadcast_to(bias2.reshape(1, fmid), (8, fmid))
    zeros128 = jnp.zeros((npad, 128), jnp.float32)
    batch3 = batch.reshape(10, 1, n // 10)

    gather = _make_gather(n, 128, e)
    scat = _make_scatter(n, e)

    a3d = edge_attr.reshape(e // 128, 1, 128)
    A1t3 = jnp.broadcast_to(A1m.T[:, :, None], (fmid, fin, 128))
    b1t3 = jnp.broadcast_to(b1m.T[:, :, None], (fmid, fin, 128))
    A2t3 = jnp.broadcast_to(A2m.T[:, :, None], (fmid, fmid, 128))
    b2t3 = jnp.broadcast_to(b2m.T[:, :, None], (fmid, fmid, 128))

    xs = gather(x, src)
    msgs1 = _edge_messages(a3d, xs, A1t3, b1t3, fin)
    p1 = scat(msgs1, dst, zeros128)
    h1p = _combine(x, root1, p1[0, :n], p1[1, :n], bias1_8)

    hs = gather(h1p, src)
    msgs2 = _edge_messages(a3d, hs, A2t3, b2t3, fmid)
    p2 = scat(msgs2, dst, zeros128)
    h2p = _combine(h1p, root2, p2[0, :n], p2[1, :n], bias2_8)

    pooled = _pool(batch3, h2p)
    return pooled[:10, :fmid]


# speedup vs baseline: 3.0252x; 1.0734x over previous
import functools

import jax
import jax.numpy as jnp
from jax import lax
from jax.experimental import pallas as pl
from jax.experimental.pallas import tpu as pltpu
from jax.experimental.pallas import tpu_sc as plsc

_INFO = plsc.get_sparse_core_info()
_NC = _INFO.num_cores
_NS = _INFO.num_subcores
_NW = _NC * _NS
_LC = 128


def _make_gather(n_nodes, feat, n_edges):
    nchunk = n_edges // _LC
    steps = (nchunk + _NW - 1) // _NW
    mesh = plsc.VectorSubcoreMesh(core_axis_name="c", subcore_axis_name="s")

    @functools.partial(
        pl.kernel,
        out_type=jax.ShapeDtypeStruct((n_edges, feat), jnp.float32),
        mesh=mesh,
        scratch_types=[
            pltpu.VMEM((_LC,), jnp.int32),
            pltpu.VMEM((_LC, feat), jnp.float32),
            pltpu.SemaphoreType.DMA,
        ],
    )
    def gather(table_hbm, src_hbm, out_hbm, idx_v, rows_v, sem):
        cid = lax.axis_index("c")
        sid = lax.axis_index("s")
        wid = sid * _NC + cid

        def body(i, carry):
            c = wid + i * _NW

            @pl.when(c < nchunk)
            def _():
                pltpu.sync_copy(src_hbm.at[c, 0], idx_v)
                pltpu.async_copy(table_hbm.at[idx_v], rows_v, sem).wait()
                pltpu.sync_copy(rows_v, out_hbm.at[pl.ds(c * _LC, _LC)])

            return carry

        lax.fori_loop(0, steps, body, 0)

    return gather


def _pad_rows(n):
    q = 8 * _NS
    return (n + q - 1) // q * q


def _make_scatter(n_nodes, n_edges):
    nchunk = n_edges // _LC
    steps = (nchunk + _NW - 1) // _NW
    npad = _pad_rows(n_nodes)
    rpt = npad // _NS
    mesh = plsc.VectorSubcoreMesh(core_axis_name="c", subcore_axis_name="s")

    @functools.partial(
        pl.kernel,
        out_type=jax.ShapeDtypeStruct((_NC, npad, 128), jnp.float32),
        mesh=mesh,
        scratch_types=[
            pltpu.VMEM((_LC,), jnp.int32),
            pltpu.VMEM((_LC, 128), jnp.float32),
            pltpu.VMEM_SHARED((npad, 128), jnp.float32),
            pltpu.SemaphoreType.DMA,
        ],
    )
    def scatter(rows_hbm, dst_hbm, zeros_hbm, out_hbm, idx_v, rows_v, agg_sh, sem):
        cid = lax.axis_index("c")
        sid = lax.axis_index("s")
        wid = sid * _NC + cid

        pltpu.sync_copy(zeros_hbm.at[pl.ds(sid * rpt, rpt)],
                        agg_sh.at[pl.ds(sid * rpt, rpt)])
        plsc.subcore_barrier()

        def body(i, carry):
            c = wid + i * _NW

            @pl.when(c < nchunk)
            def _():
                pltpu.sync_copy(dst_hbm.at[c, 0], idx_v)
                pltpu.sync_copy(rows_hbm.at[pl.ds(c * _LC, _LC)], rows_v)
                pltpu.sync_copy(rows_v, agg_sh.at[idx_v], add=True)

            return carry

        lax.fori_loop(0, steps, body, 0)
        plsc.subcore_barrier()

        pltpu.sync_copy(agg_sh.at[pl.ds(sid * rpt, rpt)],
                        out_hbm.at[cid, pl.ds(sid * rpt, rpt)])

    return scatter


def _edge_messages(a3, xs, At3, bt3, fin, eb=256):
    e = xs.shape[0]
    fout = At3.shape[0]

    def body(a_ref, xs_ref, A_ref, b_ref, o_ref):
        av = a_ref[...]
        xst = xs_ref[...][:, :fin].T
        w = jnp.maximum(av * A_ref[...] + b_ref[...], 0.0)
        msg_t = jnp.sum(xst[None, :, :] * w, axis=1)
        full = jnp.concatenate(
            [msg_t,
             jnp.ones((16, eb), jnp.float32),
             jnp.zeros((128 - fout - 16, eb), jnp.float32)], axis=0)
        o_ref[...] = full.T

    return pl.pallas_call(
        body,
        grid=(e // eb,),
        in_specs=[
            pl.BlockSpec((1, 1, eb), lambda i: (i, 0, 0)),
            pl.BlockSpec((eb, xs.shape[1]), lambda i: (i, 0)),
            pl.BlockSpec((fout, fin, eb), lambda i: (0, 0, 0)),
            pl.BlockSpec((fout, fin, eb), lambda i: (0, 0, 0)),
        ],
        out_specs=pl.BlockSpec((eb, 128), lambda i: (i, 0)),
        out_shape=jax.ShapeDtypeStruct((e, 128), jnp.float32),
    )(a3, xs, At3, bt3)


def _combine(x, root, p0, p1, bias8, nb=1000):
    n, fin = x.shape
    fout = root.shape[1]

    def body(x_ref, r_ref, p0_ref, p1_ref, b_ref, o_ref):
        p = p0_ref[...] + p1_ref[...]
        agg = p[:, :fout]
        deg = jnp.maximum(p[:, fout:fout + 1], 1.0)
        h = jnp.dot(x_ref[...][:, :fin], r_ref[...],
                    preferred_element_type=jnp.float32)
        val = jnp.maximum(h + agg / deg + b_ref[0:1, :], 0.0)
        o_ref[...] = jnp.concatenate(
            [val, jnp.zeros((val.shape[0], 128 - fout), jnp.float32)], axis=1)

    return pl.pallas_call(
        body,
        grid=(n // nb,),
        in_specs=[
            pl.BlockSpec((nb, x.shape[1]), lambda i: (i, 0)),
            pl.BlockSpec((fin, fout), lambda i: (0, 0)),
            pl.BlockSpec((nb, 128), lambda i: (i, 0)),
            pl.BlockSpec((nb, 128), lambda i: (i, 0)),
            pl.BlockSpec((8, fout), lambda i: (0, 0)),
        ],
        out_specs=pl.BlockSpec((nb, 128), lambda i: (i, 0)),
        out_shape=jax.ShapeDtypeStruct((n, 128), jnp.float32),
    )(x, root, p0, p1, bias8)


def _pool(batch3, h, nb=1000):
    n = h.shape[0]
    grid = n // nb

    def body(b_ref, h_ref, o_ref, s_acc, c_acc):
        i = pl.program_id(0)

        @pl.when(i == 0)
        def _():
            s_acc[...] = jnp.zeros_like(s_acc)
            c_acc[...] = jnp.zeros_like(c_acc)

        bb = b_ref[0, 0, :]
        gid = lax.broadcasted_iota(jnp.int32, (16, nb), 0)
        m = (gid == bb[None, :]).astype(jnp.float32)
        s_acc[...] += jnp.dot(m, h_ref[...],
                              preferred_element_type=jnp.float32)
        c_acc[...] += jnp.broadcast_to(
            jnp.sum(m, axis=1, keepdims=True), c_acc.shape)

        @pl.when(i == grid - 1)
        def _():
            o_ref[...] = s_acc[...] / jnp.maximum(c_acc[...], 1.0)

    return pl.pallas_call(
        body,
        grid=(grid,),
        in_specs=[
            pl.BlockSpec((1, 1, nb), lambda i: (i, 0, 0)),
            pl.BlockSpec((nb, 128), lambda i: (i, 0)),
        ],
        out_specs=pl.BlockSpec((16, 128), lambda i: (0, 0)),
        out_shape=jax.ShapeDtypeStruct((16, 128), jnp.float32),
        scratch_shapes=[
            pltpu.VMEM((16, 128), jnp.float32),
            pltpu.VMEM((16, 128), jnp.float32),
        ],
    )(batch3, h)


def kernel(x, edge_index, edge_attr, batch, A1, b1, root1, bias1,
           A2, b2, root2, bias2):
    n, fin = x.shape
    e = edge_attr.shape[0]
    fmid = root1.shape[1]

    src = edge_index[0].reshape(e // _LC, 1, _LC)
    dst = edge_index[1].reshape(e // _LC, 1, _LC)
    npad = _pad_rows(n)
    A1m = A1.reshape(fin, fmid)
    b1m = b1.reshape(fin, fmid)
    A2m = A2.reshape(fmid, fmid)
    b2m = b2.reshape(fmid, fmid)
    bias1_8 = jnp.broadcast_to(bias1.reshape(1, fmid), (8, fmid))
    bias2_8 = jnp.broadcast_to(bias2.reshape(1, fmid), (8, fmid))
    zeros128 = jnp.zeros((npad, 128), jnp.float32)
    batch3 = batch.reshape(10, 1, n // 10)

    gather = _make_gather(n, 128, e)
    scat = _make_scatter(n, e)

    ebm = 256
    a3d = edge_attr.reshape(e // ebm, 1, ebm)
    A1t3 = jnp.broadcast_to(A1m.T[:, :, None], (fmid, fin, ebm))
    b1t3 = jnp.broadcast_to(b1m.T[:, :, None], (fmid, fin, ebm))
    A2t3 = jnp.broadcast_to(A2m.T[:, :, None], (fmid, fmid, ebm))
    b2t3 = jnp.broadcast_to(b2m.T[:, :, None], (fmid, fmid, ebm))

    xs = gather(x, src)
    msgs1 = _edge_messages(a3d, xs, A1t3, b1t3, fin)
    p1 = scat(msgs1, dst, zeros128)
    h1p = _combine(x, root1, p1[0, :n], p1[1, :n], bias1_8)

    hs = gather(h1p, src)
    msgs2 = _edge_messages(a3d, hs, A2t3, b2t3, fmid)
    p2 = scat(msgs2, dst, zeros128)
    h2p = _combine(h1p, root2, p2[0, :n], p2[1, :n], bias2_8)

    pooled = _pool(batch3, h2p)
    return pooled[:10, :fmid]
